# baseline (speedup 1.0000x reference)
import jax
import jax.numpy as jnp
from jax.experimental import pallas as pl

N_TOKENS = 16384
E_DIM = 256
N_CODES = 1024
BLK = 4096
BETA = 0.25


def _rvq_block(x_ref, cb0_ref, cb1_ref, cb2_ref, xq_ref, idx_ref, loss_ref):
    step = pl.program_id(0)
    r = x_ref[...]
    s = jnp.sum(r * r, axis=1, keepdims=True)
    xq = jnp.zeros_like(r)
    iota = jax.lax.broadcasted_iota(jnp.int32, (BLK, N_CODES), 1)
    loss_sums = []
    idx_cols = []
    for cb_ref in (cb0_ref, cb1_ref, cb2_ref):
        cb = cb_ref[...]
        csq = jnp.sum(cb * cb, axis=1)
        p = jax.lax.dot_general(r, cb, (((1,), (1,)), ((), ())),
                                preferred_element_type=jnp.float32)
        d = (s - 2.0 * p) + csq[None, :]
        m = jnp.min(d, axis=1, keepdims=True)
        idx = jnp.min(jnp.where(d == m, iota, N_CODES), axis=1, keepdims=True)
        oh = (iota == idx).astype(jnp.float32)
        cb_hi = cb.astype(jnp.bfloat16).astype(jnp.float32)
        cb_lo = cb - cb_hi
        q_hi = jax.lax.dot_general(oh, cb_hi, (((1,), (0,)), ((), ())),
                                   preferred_element_type=jnp.float32)
        q_lo = jax.lax.dot_general(oh, cb_lo, (((1,), (0,)), ((), ())),
                                   preferred_element_type=jnp.float32)
        q = q_hi + q_lo
        x_res = r + (q - r)
        r = r - x_res
        xq = xq + x_res
        s = jnp.sum(r * r, axis=1, keepdims=True)
        loss_sums.append(jnp.sum(s))
        idx_cols.append(idx)
    xq_ref[...] = xq
    col = jax.lax.broadcasted_iota(jnp.int32, (BLK, 128), 1)
    idx_ref[...] = jnp.where(col == 0, idx_cols[0],
                             jnp.where(col == 1, idx_cols[1],
                                       jnp.where(col == 2, idx_cols[2], 0)))
    row = jax.lax.broadcasted_iota(jnp.int32, (8, 128), 0)
    tile = jnp.where(row == 0, loss_sums[0],
                     jnp.where(row == 1, loss_sums[1],
                               jnp.where(row == 2, loss_sums[2], 0.0)))

    @pl.when(step == 0)
    def _():
        loss_ref[...] = jnp.zeros_like(loss_ref)

    loss_ref[...] += tile


def kernel(x, cb0, cb1, cb2):
    grid = N_TOKENS // BLK
    xq, idx, loss = pl.pallas_call(
        _rvq_block,
        grid=(grid,),
        in_specs=[
            pl.BlockSpec((BLK, E_DIM), lambda i: (i, 0)),
            pl.BlockSpec((N_CODES, E_DIM), lambda i: (0, 0)),
            pl.BlockSpec((N_CODES, E_DIM), lambda i: (0, 0)),
            pl.BlockSpec((N_CODES, E_DIM), lambda i: (0, 0)),
        ],
        out_specs=[
            pl.BlockSpec((BLK, E_DIM), lambda i: (i, 0)),
            pl.BlockSpec((BLK, 128), lambda i: (i, 0)),
            pl.BlockSpec((8, 128), lambda i: (0, 0)),
        ],
        out_shape=[
            jax.ShapeDtypeStruct((N_TOKENS, E_DIM), jnp.float32),
            jax.ShapeDtypeStruct((N_TOKENS, 128), jnp.int32),
            jax.ShapeDtypeStruct((8, 128), jnp.float32),
        ],
    )(x, cb0, cb1, cb2)
    sums = loss[:, 0]
    mean_loss = ((1.0 + BETA) / (3.0 * N_TOKENS * E_DIM)) * (
        sums[0] + sums[1] + sums[2])
    return xq, mean_loss, idx[:, :3]


# speedup vs baseline: 1.1957x; 1.1957x over previous
import jax
import jax.numpy as jnp
from jax.experimental import pallas as pl

N_TOKENS = 16384
E_DIM = 256
N_CODES = 1024
BLK = 1024
BETA = 0.25


def _rvq_block(x_ref, cb0_ref, cb1_ref, cb2_ref, xq_ref, idx_ref, loss_ref):
    step = pl.program_id(0)
    r = x_ref[...]
    s = jnp.sum(r * r, axis=1, keepdims=True)
    xq = jnp.zeros_like(r)
    iota = jax.lax.broadcasted_iota(jnp.int32, (BLK, N_CODES), 1)
    loss_sums = []
    idx_cols = []
    for cb_ref in (cb0_ref, cb1_ref, cb2_ref):
        cb = cb_ref[...]
        csq = jnp.sum(cb * cb, axis=1)
        p = jax.lax.dot_general(r, cb, (((1,), (1,)), ((), ())),
                                preferred_element_type=jnp.float32)
        d = (s - 2.0 * p) + csq[None, :]
        m = jnp.min(d, axis=1, keepdims=True)
        idx = jnp.min(jnp.where(d == m, iota, N_CODES), axis=1, keepdims=True)
        oh = (iota == idx).astype(jnp.float32)
        cb_hi = cb.astype(jnp.bfloat16).astype(jnp.float32)
        cb_lo = cb - cb_hi
        q_hi = jax.lax.dot_general(oh, cb_hi, (((1,), (0,)), ((), ())),
                                   preferred_element_type=jnp.float32)
        q_lo = jax.lax.dot_general(oh, cb_lo, (((1,), (0,)), ((), ())),
                                   preferred_element_type=jnp.float32)
        q = q_hi + q_lo
        x_res = r + (q - r)
        r = r - x_res
        xq = xq + x_res
        s = jnp.sum(r * r, axis=1, keepdims=True)
        loss_sums.append(jnp.sum(s))
        idx_cols.append(idx)
    xq_ref[...] = xq
    col = jax.lax.broadcasted_iota(jnp.int32, (BLK, 128), 1)
    idx_ref[...] = jnp.where(col == 0, idx_cols[0],
                             jnp.where(col == 1, idx_cols[1],
                                       jnp.where(col == 2, idx_cols[2], 0)))
    row = jax.lax.broadcasted_iota(jnp.int32, (8, 128), 0)
    tile = jnp.where(row == 0, loss_sums[0],
                     jnp.where(row == 1, loss_sums[1],
                               jnp.where(row == 2, loss_sums[2], 0.0)))

    @pl.when(step == 0)
    def _():
        loss_ref[...] = jnp.zeros_like(loss_ref)

    loss_ref[...] += tile


def kernel(x, cb0, cb1, cb2):
    grid = N_TOKENS // BLK
    xq, idx, loss = pl.pallas_call(
        _rvq_block,
        grid=(grid,),
        in_specs=[
            pl.BlockSpec((BLK, E_DIM), lambda i: (i, 0)),
            pl.BlockSpec((N_CODES, E_DIM), lambda i: (0, 0)),
            pl.BlockSpec((N_CODES, E_DIM), lambda i: (0, 0)),
            pl.BlockSpec((N_CODES, E_DIM), lambda i: (0, 0)),
        ],
        out_specs=[
            pl.BlockSpec((BLK, E_DIM), lambda i: (i, 0)),
            pl.BlockSpec((BLK, 128), lambda i: (i, 0)),
            pl.BlockSpec((8, 128), lambda i: (0, 0)),
        ],
        out_shape=[
            jax.ShapeDtypeStruct((N_TOKENS, E_DIM), jnp.float32),
            jax.ShapeDtypeStruct((N_TOKENS, 128), jnp.int32),
            jax.ShapeDtypeStruct((8, 128), jnp.float32),
        ],
    )(x, cb0, cb1, cb2)
    sums = loss[:, 0]
    mean_loss = ((1.0 + BETA) / (3.0 * N_TOKENS * E_DIM)) * (
        sums[0] + sums[1] + sums[2])
    return xq, mean_loss, idx[:, :3]


# speedup vs baseline: 1.2399x; 1.0370x over previous
import jax
import jax.numpy as jnp
from jax.experimental import pallas as pl

N_TOKENS = 16384
E_DIM = 256
N_CODES = 1024
BLK = 2048
BETA = 0.25


def _rvq_block(x_ref, cb0_ref, cb1_ref, cb2_ref, xq_ref, idx_ref, loss_ref):
    step = pl.program_id(0)
    r = x_ref[...]
    s = jnp.sum(r * r, axis=1, keepdims=True)
    xq = jnp.zeros_like(r)
    iota = jax.lax.broadcasted_iota(jnp.int32, (BLK, N_CODES), 1)
    loss_sums = []
    idx_cols = []
    for cb_ref in (cb0_ref, cb1_ref, cb2_ref):
        cb = cb_ref[...]
        csq = jnp.sum(cb * cb, axis=1)
        p = jax.lax.dot_general(r, cb, (((1,), (1,)), ((), ())),
                                preferred_element_type=jnp.float32)
        d = (s - 2.0 * p) + csq[None, :]
        m = jnp.min(d, axis=1, keepdims=True)
        idx = jnp.min(jnp.where(d == m, iota, N_CODES), axis=1, keepdims=True)
        oh = (iota == idx).astype(jnp.bfloat16)
        cb_hi = cb.astype(jnp.bfloat16)
        cb_lo = (cb - cb_hi.astype(jnp.float32)).astype(jnp.bfloat16)
        q_hi = jax.lax.dot_general(oh, cb_hi, (((1,), (0,)), ((), ())),
                                   preferred_element_type=jnp.float32)
        q_lo = jax.lax.dot_general(oh, cb_lo, (((1,), (0,)), ((), ())),
                                   preferred_element_type=jnp.float32)
        q = q_hi + q_lo
        x_res = r + (q - r)
        r = r - x_res
        xq = xq + x_res
        s = jnp.sum(r * r, axis=1, keepdims=True)
        loss_sums.append(jnp.sum(s))
        idx_cols.append(idx)
    xq_ref[...] = xq
    col = jax.lax.broadcasted_iota(jnp.int32, (BLK, 128), 1)
    idx_ref[...] = jnp.where(col == 0, idx_cols[0],
                             jnp.where(col == 1, idx_cols[1],
                                       jnp.where(col == 2, idx_cols[2], 0)))
    row = jax.lax.broadcasted_iota(jnp.int32, (8, 128), 0)
    tile = jnp.where(row == 0, loss_sums[0],
                     jnp.where(row == 1, loss_sums[1],
                               jnp.where(row == 2, loss_sums[2], 0.0)))

    @pl.when(step == 0)
    def _():
        loss_ref[...] = jnp.zeros_like(loss_ref)

    loss_ref[...] += tile


def kernel(x, cb0, cb1, cb2):
    grid = N_TOKENS // BLK
    xq, idx, loss = pl.pallas_call(
        _rvq_block,
        grid=(grid,),
        in_specs=[
            pl.BlockSpec((BLK, E_DIM), lambda i: (i, 0)),
            pl.BlockSpec((N_CODES, E_DIM), lambda i: (0, 0)),
            pl.BlockSpec((N_CODES, E_DIM), lambda i: (0, 0)),
            pl.BlockSpec((N_CODES, E_DIM), lambda i: (0, 0)),
        ],
        out_specs=[
            pl.BlockSpec((BLK, E_DIM), lambda i: (i, 0)),
            pl.BlockSpec((BLK, 128), lambda i: (i, 0)),
            pl.BlockSpec((8, 128), lambda i: (0, 0)),
        ],
        out_shape=[
            jax.ShapeDtypeStruct((N_TOKENS, E_DIM), jnp.float32),
            jax.ShapeDtypeStruct((N_TOKENS, 128), jnp.int32),
            jax.ShapeDtypeStruct((8, 128), jnp.float32),
        ],
    )(x, cb0, cb1, cb2)
    sums = loss[:, 0]
    mean_loss = ((1.0 + BETA) / (3.0 * N_TOKENS * E_DIM)) * (
        sums[0] + sums[1] + sums[2])
    return xq, mean_loss, idx[:, :3]


# speedup vs baseline: 1.2516x; 1.0094x over previous
import jax
import jax.numpy as jnp
from jax.experimental import pallas as pl

N_TOKENS = 16384
E_DIM = 256
N_CODES = 1024
BLK = 2048
BETA = 0.25


def _rvq_block(x_ref, cb0_ref, cb1_ref, cb2_ref, xq_ref, idx_ref, loss_ref):
    step = pl.program_id(0)
    r = x_ref[...]
    s = jnp.sum(r * r, axis=1, keepdims=True)
    xq = jnp.zeros_like(r)
    iota = jax.lax.broadcasted_iota(jnp.int32, (BLK, N_CODES), 1)
    loss_sums = []
    idx_cols = []
    for cb_ref in (cb0_ref, cb1_ref, cb2_ref):
        cb = cb_ref[...]
        csq = jnp.sum(cb * cb, axis=1)
        p = jax.lax.dot_general(r, cb, (((1,), (1,)), ((), ())),
                                preferred_element_type=jnp.float32)
        d = (s - 2.0 * p) + csq[None, :]
        m = jnp.min(d, axis=1, keepdims=True)
        idx = jnp.min(jnp.where(d == m, iota, N_CODES), axis=1, keepdims=True)
        oh = (iota == idx).astype(jnp.float32)
        cb_hi = cb.astype(jnp.bfloat16).astype(jnp.float32)
        cb_lo = cb - cb_hi
        q_hi = jax.lax.dot_general(oh, cb_hi, (((1,), (0,)), ((), ())),
                                   preferred_element_type=jnp.float32)
        q_lo = jax.lax.dot_general(oh, cb_lo, (((1,), (0,)), ((), ())),
                                   preferred_element_type=jnp.float32)
        q = q_hi + q_lo
        x_res = r + (q - r)
        r = r - x_res
        xq = xq + x_res
        s = jnp.sum(r * r, axis=1, keepdims=True)
        loss_sums.append(jnp.sum(s))
        idx_cols.append(idx)
    xq_ref[...] = xq
    col = jax.lax.broadcasted_iota(jnp.int32, (BLK, 128), 1)
    idx_ref[...] = jnp.where(col == 0, idx_cols[0],
                             jnp.where(col == 1, idx_cols[1],
                                       jnp.where(col == 2, idx_cols[2], 0)))
    row = jax.lax.broadcasted_iota(jnp.int32, (8, 128), 0)
    tile = jnp.where(row == 0, loss_sums[0],
                     jnp.where(row == 1, loss_sums[1],
                               jnp.where(row == 2, loss_sums[2], 0.0)))

    @pl.when(step == 0)
    def _():
        loss_ref[...] = jnp.zeros_like(loss_ref)

    loss_ref[...] += tile


def kernel(x, cb0, cb1, cb2):
    grid = N_TOKENS // BLK
    xq, idx, loss = pl.pallas_call(
        _rvq_block,
        grid=(grid,),
        in_specs=[
            pl.BlockSpec((BLK, E_DIM), lambda i: (i, 0)),
            pl.BlockSpec((N_CODES, E_DIM), lambda i: (0, 0)),
            pl.BlockSpec((N_CODES, E_DIM), lambda i: (0, 0)),
            pl.BlockSpec((N_CODES, E_DIM), lambda i: (0, 0)),
        ],
        out_specs=[
            pl.BlockSpec((BLK, E_DIM), lambda i: (i, 0)),
            pl.BlockSpec((BLK, 128), lambda i: (i, 0)),
            pl.BlockSpec((8, 128), lambda i: (0, 0)),
        ],
        out_shape=[
            jax.ShapeDtypeStruct((N_TOKENS, E_DIM), jnp.float32),
            jax.ShapeDtypeStruct((N_TOKENS, 128), jnp.int32),
            jax.ShapeDtypeStruct((8, 128), jnp.float32),
        ],
    )(x, cb0, cb1, cb2)
    sums = loss[:, 0]
    mean_loss = ((1.0 + BETA) / (3.0 * N_TOKENS * E_DIM)) * (
        sums[0] + sums[1] + sums[2])
    return xq, mean_loss, idx[:, :3]
